# baseline (device time: 14001 ns/iter reference)
import jax
import jax.numpy as jnp
from jax import lax
from jax.experimental import pallas as pl
from jax.experimental.pallas import tpu as pltpu

N_DEV = 4
B, SQ, D = 2, 128, 512
HQ_LOCAL, DH = 8, 64
G = 2
HPG = 4


def kernel(x, Wq, Wo, Wk, Wv):
    x = pltpu.with_memory_space_constraint(x, pltpu.MemorySpace.HBM)
    Wq = pltpu.with_memory_space_constraint(Wq, pltpu.MemorySpace.HBM)
    Wo = pltpu.with_memory_space_constraint(Wo, pltpu.MemorySpace.HBM)
    Wk = pltpu.with_memory_space_constraint(Wk, pltpu.MemorySpace.HBM)
    Wv = pltpu.with_memory_space_constraint(Wv, pltpu.MemorySpace.HBM)

    def body(x_hbm, wq_hbm, wo_hbm, wk_hbm, wv_hbm, out_hbm,
             xv, wqv, wov, wkv, wvv, outv,
             mine_ref, fromL_ref, fromR_ref, fromD_ref,
             in_sems, out_sems, send_sems, recv_sems):
        me = lax.axis_index("i")
        left = lax.rem(me + N_DEV - 1, N_DEV)
        right = lax.rem(me + 1, N_DEV)

        barrier = pltpu.get_barrier_semaphore()
        for nbr in (left, right):
            pl.semaphore_signal(barrier, inc=1, device_id=(nbr,),
                                device_id_type=pl.DeviceIdType.MESH)

        cx = pltpu.make_async_copy(x_hbm, xv, in_sems.at[0])
        cq = pltpu.make_async_copy(wq_hbm, wqv, in_sems.at[1])
        co = pltpu.make_async_copy(wo_hbm, wov, in_sems.at[2])
        ck = pltpu.make_async_copy(
            wk_hbm.at[:, pl.ds(me * G * DH, G * DH)], wkv, in_sems.at[3])
        cv = pltpu.make_async_copy(
            wv_hbm.at[:, pl.ds(me * G * DH, G * DH)], wvv, in_sems.at[4])
        cx.start()
        cq.start()
        co.start()
        ck.start()
        cv.start()

        cx.wait()
        cq.wait()
        ck.wait()
        cv.wait()
        co.wait()
        wkv16 = jnp.concatenate(
            [wkv[...].astype(jnp.bfloat16), wvv[...].astype(jnp.bfloat16)],
            axis=1)
        wq16 = wqv[...].astype(jnp.bfloat16)
        wo16 = wov[...].astype(jnp.bfloat16)

        def compute_partial(b):
            xb = xv[b].astype(jnp.bfloat16)
            qb = (jnp.dot(xb, wq16,
                          preferred_element_type=jnp.float32)
                  * 0.125).astype(jnp.bfloat16)
            kvb = jnp.dot(xb, wkv16,
                          preferred_element_type=jnp.float32
                          ).astype(jnp.bfloat16)
            kb = kvb[:, :G * DH]
            vb = kvb[:, G * DH:]
            head_outs = [None] * HQ_LOCAL
            for g in range(G):
                qstack = jnp.concatenate(
                    [qb[:, (g * HPG + i) * DH:(g * HPG + i + 1) * DH]
                     for i in range(HPG)], axis=0)
                kh = kb[:, g * DH:(g + 1) * DH]
                vh = vb[:, g * DH:(g + 1) * DH]
                s = lax.dot_general(qstack, kh, (((1,), (1,)), ((), ())),
                                    preferred_element_type=jnp.float32)
                p = jnp.exp(s)
                denom = jnp.sum(p, axis=-1, keepdims=True)
                p = (p / denom).astype(jnp.bfloat16)
                o = jnp.dot(p, vh, preferred_element_type=jnp.float32
                            ).astype(jnp.bfloat16)
                for i in range(HPG):
                    head_outs[g * HPG + i] = o[i * SQ:(i + 1) * SQ, :]
            attn_b = jnp.concatenate(head_outs, axis=1)
            return jnp.dot(attn_b, wo16,
                           preferred_element_type=jnp.float32)

        def phase_a(b):
            d_r = pltpu.make_async_remote_copy(
                src_ref=mine_ref.at[b], dst_ref=fromL_ref.at[b],
                send_sem=send_sems.at[2 * b], recv_sem=recv_sems.at[2 * b],
                device_id=(right,), device_id_type=pl.DeviceIdType.MESH,
            )
            d_l = pltpu.make_async_remote_copy(
                src_ref=mine_ref.at[b], dst_ref=fromR_ref.at[b],
                send_sem=send_sems.at[2 * b + 1],
                recv_sem=recv_sems.at[2 * b + 1],
                device_id=(left,), device_id_type=pl.DeviceIdType.MESH,
            )
            d_r.start()
            d_l.start()
            return d_r, d_l

        p0 = compute_partial(0)
        mine_ref[0] = p0.astype(jnp.bfloat16)
        pl.semaphore_wait(barrier, 2)
        d_ar0, d_al0 = phase_a(0)

        p1 = compute_partial(1)
        mine_ref[1] = p1.astype(jnp.bfloat16)
        d_ar1, d_al1 = phase_a(1)

        d_ar0.wait_recv()
        d_br = pltpu.make_async_remote_copy(
            src_ref=fromL_ref.at[0], dst_ref=fromD_ref.at[0],
            send_sem=send_sems.at[4], recv_sem=recv_sems.at[4],
            device_id=(right,), device_id_type=pl.DeviceIdType.MESH,
        )
        d_br.start()
        d_al0.wait_recv()
        acc0 = (p0 + fromL_ref[0].astype(jnp.float32)
                + fromR_ref[0].astype(jnp.float32))

        d_al1.wait_recv()
        d_bl = pltpu.make_async_remote_copy(
            src_ref=fromR_ref.at[1], dst_ref=fromD_ref.at[1],
            send_sem=send_sems.at[5], recv_sem=recv_sems.at[5],
            device_id=(left,), device_id_type=pl.DeviceIdType.MESH,
        )
        d_bl.start()
        d_ar1.wait_recv()
        acc1 = (p1 + fromL_ref[1].astype(jnp.float32)
                + fromR_ref[1].astype(jnp.float32))

        d_br.wait_recv()
        outv[0] = acc0 + fromD_ref[0].astype(jnp.float32)
        o0 = pltpu.make_async_copy(outv.at[0], out_hbm.at[0], out_sems.at[0])
        o0.start()

        d_bl.wait_recv()
        outv[1] = acc1 + fromD_ref[1].astype(jnp.float32)
        o1 = pltpu.make_async_copy(outv.at[1], out_hbm.at[1], out_sems.at[1])
        o1.start()

        o0.wait()
        o1.wait()
        d_ar0.wait_send()
        d_al0.wait_send()
        d_ar1.wait_send()
        d_al1.wait_send()
        d_br.wait_send()
        d_bl.wait_send()

    return pl.pallas_call(
        body,
        out_shape=jax.ShapeDtypeStruct((B, SQ, D), jnp.float32),
        in_specs=[pl.BlockSpec(memory_space=pltpu.MemorySpace.HBM)] * 5,
        out_specs=pl.BlockSpec(memory_space=pltpu.MemorySpace.HBM),
        scratch_shapes=[
            pltpu.VMEM((B, SQ, D), jnp.float32),
            pltpu.VMEM((D, D), jnp.float32),
            pltpu.VMEM((D, D), jnp.float32),
            pltpu.VMEM((D, G * DH), jnp.float32),
            pltpu.VMEM((D, G * DH), jnp.float32),
            pltpu.VMEM((B, SQ, D), jnp.float32),
            pltpu.VMEM((B, SQ, D), jnp.bfloat16),
            pltpu.VMEM((B, SQ, D), jnp.bfloat16),
            pltpu.VMEM((B, SQ, D), jnp.bfloat16),
            pltpu.VMEM((B, SQ, D), jnp.bfloat16),
            pltpu.SemaphoreType.DMA((5,)),
            pltpu.SemaphoreType.DMA((2,)),
            pltpu.SemaphoreType.DMA((6,)),
            pltpu.SemaphoreType.DMA((6,)),
        ],
        compiler_params=pltpu.CompilerParams(collective_id=0),
    )(x, Wq, Wo, Wk, Wv)
